# baseline (speedup 1.0000x reference)
import functools

import jax
import jax.numpy as jnp
from jax import lax
from jax.experimental import pallas as pl
from jax.experimental.pallas import tpu as pltpu
from jax.experimental.pallas import tpu_sc as plsc

NUM_MODELS = 8
NUM_STATES = 2048
VOCAB = 1024
BATCH = 128

NUM_CORES = 2
LANES = 16

NW = 16
ROWS_PER_W = BATCH // NW


def _sc_body(states_hbm, mid_hbm, ns_hbm, ns_out,
             st_v, md_v, idx_v, ns_rows, sem_a, sem_b):
    c = lax.axis_index("c")
    s = lax.axis_index("s")
    wid = s + c * 0
    chunk = wid // 2
    half = wid % 2
    cp_st = pltpu.async_copy(
        states_hbm.at[pl.ds(chunk * LANES, LANES)], st_v, sem_a)
    cp_md = pltpu.async_copy(
        mid_hbm.at[pl.ds(chunk * LANES, LANES)], md_v, sem_b)
    cp_st.wait()
    cp_md.wait()
    idx_v[...] = md_v[...] * NUM_STATES + st_v[...]
    idx_slice = idx_v.at[pl.ds(half * ROWS_PER_W, ROWS_PER_W)]
    pltpu.async_copy(ns_hbm.at[idx_slice], ns_rows, sem_a).wait()
    pltpu.sync_copy(
        ns_rows, ns_out.at[pl.ds(wid * ROWS_PER_W, ROWS_PER_W)])


def _sc_ns(states, model_ids, ns2d):
    mesh = plsc.VectorSubcoreMesh(
        core_axis_name="c", subcore_axis_name="s", num_cores=1)
    f = pl.kernel(
        _sc_body,
        out_type=jax.ShapeDtypeStruct((BATCH, VOCAB), jnp.int32),
        mesh=mesh,
        scratch_types=(
            pltpu.VMEM((LANES,), jnp.int32),
            pltpu.VMEM((LANES,), jnp.int32),
            pltpu.VMEM((LANES,), jnp.int32),
            pltpu.VMEM((ROWS_PER_W, VOCAB), jnp.int32),
            pltpu.SemaphoreType.DMA,
            pltpu.SemaphoreType.DMA,
        ),
    )
    return f(states, model_ids, ns2d)


def _tc_body(st_ref, md_ref, al_ref, md2_ref, tbl_ref, out_ref, buf, sem):
    cps = []
    for b in range(BATCH):
        idx = md_ref[b] * NUM_STATES + st_ref[b]
        cp = pltpu.make_async_copy(
            tbl_ref.at[pl.ds(idx, 1)], buf.at[pl.ds(b, 1)], sem.at[b % 8])
        cp.start()
        cps.append(cp)
    alpha = jnp.full((BATCH, 1), 0.0, dtype=jnp.float32)
    for m in range(NUM_MODELS):
        alpha = jnp.where(md2_ref[...] == m, al_ref[m], alpha)
    for cp in cps:
        cp.wait()
    out_ref[...] = buf[...] * alpha


def _tc_scores(states, model_ids, alphas, scores2d):
    md2 = model_ids.reshape(BATCH, 1)
    return pl.pallas_call(
        _tc_body,
        out_shape=jax.ShapeDtypeStruct((BATCH, VOCAB), jnp.float32),
        in_specs=[
            pl.BlockSpec(memory_space=pltpu.SMEM),
            pl.BlockSpec(memory_space=pltpu.SMEM),
            pl.BlockSpec(memory_space=pltpu.SMEM),
            pl.BlockSpec(memory_space=pltpu.VMEM),
            pl.BlockSpec(memory_space=pltpu.HBM),
        ],
        out_specs=pl.BlockSpec(memory_space=pltpu.VMEM),
        scratch_shapes=[
            pltpu.VMEM((BATCH, VOCAB), jnp.float32),
            pltpu.SemaphoreType.DMA((8,)),
        ],
    )(states, model_ids, alphas, md2, scores2d)


@jax.jit
def _run(states, model_ids, scores2d, ns2d, alphas):
    scores = _tc_scores(states, model_ids, alphas, scores2d)
    next_states = _sc_ns(states, model_ids, ns2d)
    return scores, next_states


def kernel(states, model_ids, scores_tables, next_states_tables, alphas):
    scores2d = scores_tables.reshape(NUM_MODELS * NUM_STATES, VOCAB)
    ns2d = next_states_tables.reshape(NUM_MODELS * NUM_STATES, VOCAB)
    return _run(states, model_ids, scores2d, ns2d, alphas)


# speedup vs baseline: 1.0080x; 1.0080x over previous
import functools

import jax
import jax.numpy as jnp
from jax import lax
from jax.experimental import pallas as pl
from jax.experimental.pallas import tpu as pltpu
from jax.experimental.pallas import tpu_sc as plsc

NUM_MODELS = 8
NUM_STATES = 2048
VOCAB = 1024
BATCH = 128

NUM_CORES = 2
LANES = 16

NW = 16
ROWS_PER_W = BATCH // NW
HALF_ROWS = ROWS_PER_W // 2


_GATHER_DNUMS = lax.GatherDimensionNumbers(
    offset_dims=(), collapsed_slice_dims=(0,), start_index_map=(0,))


def _reg_gather(src, idx):
    return lax.gather(src, idx[:, None], _GATHER_DNUMS, slice_sizes=(1,),
                      mode=lax.GatherScatterMode.PROMISE_IN_BOUNDS)


def _sc_body(states_hbm, mid_hbm, ns_hbm, ns_out,
             st_v, md_v, idx_v, idx_v2, rows_a, rows_b, sem_a, sem_b, sem_c):
    c = lax.axis_index("c")
    s = lax.axis_index("s")
    wid = s + c * 0
    chunk = wid // 2
    half = wid % 2
    cp_st = pltpu.async_copy(
        states_hbm.at[pl.ds(chunk * LANES, LANES)], st_v, sem_a)
    cp_md = pltpu.async_copy(
        mid_hbm.at[pl.ds(chunk * LANES, LANES)], md_v, sem_b)
    cp_st.wait()
    cp_md.wait()
    idx16 = md_v[...] * NUM_STATES + st_v[...]
    idx_v[...] = idx16
    idx_v2[...] = _reg_gather(
        idx16, (lax.iota(jnp.int32, LANES) + HALF_ROWS) % LANES)
    base = half * ROWS_PER_W
    cp1 = pltpu.async_copy(
        ns_hbm.at[idx_v.at[pl.ds(base, HALF_ROWS)]], rows_a, sem_a)
    cp2 = pltpu.async_copy(
        ns_hbm.at[idx_v2.at[pl.ds(base, HALF_ROWS)]], rows_b, sem_b)
    cp1.wait()
    out_base = wid * ROWS_PER_W
    cpo1 = pltpu.async_copy(
        rows_a, ns_out.at[pl.ds(out_base, HALF_ROWS)], sem_c)
    cp2.wait()
    cpo2 = pltpu.async_copy(
        rows_b, ns_out.at[pl.ds(out_base + HALF_ROWS, HALF_ROWS)], sem_a)
    cpo1.wait()
    cpo2.wait()


def _sc_ns(states, model_ids, ns2d):
    mesh = plsc.VectorSubcoreMesh(
        core_axis_name="c", subcore_axis_name="s", num_cores=1)
    f = pl.kernel(
        _sc_body,
        out_type=jax.ShapeDtypeStruct((BATCH, VOCAB), jnp.int32),
        mesh=mesh,
        scratch_types=(
            pltpu.VMEM((LANES,), jnp.int32),
            pltpu.VMEM((LANES,), jnp.int32),
            pltpu.VMEM((LANES,), jnp.int32),
            pltpu.VMEM((LANES,), jnp.int32),
            pltpu.VMEM((HALF_ROWS, VOCAB), jnp.int32),
            pltpu.VMEM((HALF_ROWS, VOCAB), jnp.int32),
            pltpu.SemaphoreType.DMA,
            pltpu.SemaphoreType.DMA,
            pltpu.SemaphoreType.DMA,
        ),
    )
    return f(states, model_ids, ns2d)


def _tc_body(st_ref, md_ref, al_ref, md2_ref, tbl_ref, out_ref, buf, sem):
    cps = []
    for b in range(BATCH):
        idx = md_ref[b] * NUM_STATES + st_ref[b]
        cp = pltpu.make_async_copy(
            tbl_ref.at[pl.ds(idx, 1)], buf.at[pl.ds(b, 1)], sem.at[b % 8])
        cp.start()
        cps.append(cp)
    alpha = jnp.full((BATCH, 1), 0.0, dtype=jnp.float32)
    for m in range(NUM_MODELS):
        alpha = jnp.where(md2_ref[...] == m, al_ref[m], alpha)
    for cp in cps:
        cp.wait()
    out_ref[...] = buf[...] * alpha


def _tc_scores(states, model_ids, alphas, scores2d):
    md2 = model_ids.reshape(BATCH, 1)
    return pl.pallas_call(
        _tc_body,
        out_shape=jax.ShapeDtypeStruct((BATCH, VOCAB), jnp.float32),
        in_specs=[
            pl.BlockSpec(memory_space=pltpu.SMEM),
            pl.BlockSpec(memory_space=pltpu.SMEM),
            pl.BlockSpec(memory_space=pltpu.SMEM),
            pl.BlockSpec(memory_space=pltpu.VMEM),
            pl.BlockSpec(memory_space=pltpu.HBM),
        ],
        out_specs=pl.BlockSpec(memory_space=pltpu.VMEM),
        scratch_shapes=[
            pltpu.VMEM((BATCH, VOCAB), jnp.float32),
            pltpu.SemaphoreType.DMA((8,)),
        ],
    )(states, model_ids, alphas, md2, scores2d)


@jax.jit
def _run(states, model_ids, scores2d, ns2d, alphas):
    scores = _tc_scores(states, model_ids, alphas, scores2d)
    next_states = _sc_ns(states, model_ids, ns2d)
    return scores, next_states


def kernel(states, model_ids, scores_tables, next_states_tables, alphas):
    scores2d = scores_tables.reshape(NUM_MODELS * NUM_STATES, VOCAB)
    ns2d = next_states_tables.reshape(NUM_MODELS * NUM_STATES, VOCAB)
    return _run(states, model_ids, scores2d, ns2d, alphas)
